# Initial kernel scaffold; baseline (speedup 1.0000x reference)
import jax
import jax.numpy as jnp
from jax.experimental import pallas as pl


def kernel(node_features, edge_index, edge_features, W_n, b_n, W_e, b_e, Wm1, bm1, Wm2, bm2, Wu, bu, Wo, bo):
    raise NotImplementedError("write your pallas kernel here")


# speedup vs baseline: 6.5025x; 6.5025x over previous
import jax
import jax.numpy as jnp
from jax import lax
from jax.experimental import pallas as pl
from jax.experimental.pallas import tpu as pltpu
from jax.experimental.pallas import tpu_sc as plsc


N = 10000
E = 320000
D = 64
DP = 128
NC, NS, L = 2, 16, 16
TILES = NC * NS
EPT = E // TILES
SUB = 80
SUBS = 1
C = SUB * SUBS
NCHUNK = EPT // C


def _node_body(nf, wn, bn, wa, x_out, xa_out):
    h = jnp.dot(nf[...], wn[...], preferred_element_type=jnp.float32) + bn[...]
    xx = jnp.maximum(h * jax.nn.sigmoid(h), 0.0)
    x_out[...] = xx
    xad = jnp.dot(xx, wa[...], preferred_element_type=jnp.float32)
    pad = (lax.broadcasted_iota(jnp.int32, (xx.shape[0], DP - D), 1) == 0)
    xa_out[...] = jnp.concatenate([xad, pad.astype(jnp.float32)], axis=1)


def _edge_body(ef, we, be, wb, bm, ebb_out):
    h = jnp.dot(ef[...], we[...], preferred_element_type=jnp.float32) + be[...]
    ee = jnp.maximum(h * jax.nn.sigmoid(h), 0.0)
    ebb_out[...] = jnp.dot(ee, wb[...], preferred_element_type=jnp.float32) + bm[...]


def _final_body(s2, x, wm2, bm2, wua, wux, bu, wo, bo, out):
    stot = s2[0] + s2[1]
    aggr = (jnp.dot(stot[:, :D], wm2[...], preferred_element_type=jnp.float32)
            + stot[:, D:D + 1] * bm2[...])
    h = (jnp.dot(aggr, wua[...], preferred_element_type=jnp.float32)
         + jnp.dot(x[...], wux[...], preferred_element_type=jnp.float32)
         + bu[...])
    h = jnp.maximum(h * jax.nn.sigmoid(h), 0.0)
    out[...] = jnp.dot(h, wo[...], preferred_element_type=jnp.float32) + bo[...]


def _sc_edge_kernel(xa_hbm, row_hbm, col_hbm, ebb_hbm, zeros_hbm, out_hbm,
                    s_sh, idx0_v, idx1_v, ebb_v, g0_v, g1_v,
                    sem_e, sem_g):
    cid = lax.axis_index("c")
    sid = lax.axis_index("s")
    tile = cid * NS + sid
    base = tile * EPT

    @pl.when(sid == 0)
    def _():
        pltpu.sync_copy(zeros_hbm, s_sh)

    plsc.subcore_barrier()

    def chunk_body(i, carry):
        offs = base + i * C
        ce = pltpu.async_copy(ebb_hbm.at[pl.ds(offs, C)], ebb_v, sem_e)
        for k in range(SUBS):
            pltpu.sync_copy(row_hbm.at[pl.ds(offs + k * SUB, SUB)], idx0_v.at[k])
            pltpu.sync_copy(col_hbm.at[pl.ds(offs + k * SUB, SUB)], idx1_v.at[k])
        cps = []
        for k in range(SUBS):
            cps.append(pltpu.async_copy(
                xa_hbm.at[idx0_v.at[k]], g0_v.at[pl.ds(k * SUB, SUB)], sem_g))
            cps.append(pltpu.async_copy(
                xa_hbm.at[idx1_v.at[k]], g1_v.at[pl.ds(k * SUB, SUB)], sem_g))
        for cp in cps:
            cp.wait()
        ce.wait()

        def row_body(r, rc):
            for j in range(D // L):
                sl = pl.ds(j * L, L)
                eb = ebb_v[r, sl]
                u0 = g0_v[r, sl] + eb
                g0_v[r, sl] = u0 / (1.0 + jnp.exp(-u0))
                u1 = g1_v[r, sl] + eb
                g1_v[r, sl] = u1 / (1.0 + jnp.exp(-u1))
            return rc

        lax.fori_loop(0, C, row_body, 0)

        for k in range(SUBS):
            pltpu.sync_copy(g0_v.at[pl.ds(k * SUB, SUB)],
                            s_sh.at[idx1_v.at[k]], add=True)
            pltpu.sync_copy(g1_v.at[pl.ds(k * SUB, SUB)],
                            s_sh.at[idx0_v.at[k]], add=True)
        return carry

    lax.fori_loop(0, NCHUNK, chunk_body, 0)

    plsc.subcore_barrier()

    @pl.when(sid == 0)
    def _():
        pltpu.sync_copy(s_sh, out_hbm.at[cid])


def kernel(node_features, edge_index, edge_features,
           W_n, b_n, W_e, b_e, Wm1, bm1, Wm2, bm2, Wu, bu, Wo, bo):
    nd = node_features.shape[1]
    ed = edge_features.shape[1]
    og = Wo.shape[1]

    wa = Wm1[:D]
    wb = Wm1[D:]
    bn2 = b_n.reshape(1, -1)
    be2 = b_e.reshape(1, -1)
    bm1r = bm1.reshape(1, -1)
    bm2r = bm2.reshape(1, -1)
    bu2 = bu.reshape(1, -1)
    wo_pad = jnp.zeros((D, 128), jnp.float32).at[:, :og].set(Wo)
    bo_pad = jnp.zeros((1, 128), jnp.float32).at[0, :og].set(bo)

    BN = 2000
    x, xa = pl.pallas_call(
        _node_body,
        grid=(N // BN,),
        in_specs=[pl.BlockSpec((BN, nd), lambda i: (i, 0)),
                  pl.BlockSpec((nd, D), lambda i: (0, 0)),
                  pl.BlockSpec((1, D), lambda i: (0, 0)),
                  pl.BlockSpec((D, D), lambda i: (0, 0))],
        out_specs=[pl.BlockSpec((BN, D), lambda i: (i, 0)),
                   pl.BlockSpec((BN, DP), lambda i: (i, 0))],
        out_shape=[jax.ShapeDtypeStruct((N, D), jnp.float32),
                   jax.ShapeDtypeStruct((N, DP), jnp.float32)],
    )(node_features, W_n, bn2, wa)

    BE = 8000
    ebb = pl.pallas_call(
        _edge_body,
        grid=(E // BE,),
        in_specs=[pl.BlockSpec((BE, ed), lambda i: (i, 0)),
                  pl.BlockSpec((ed, ed), lambda i: (0, 0)),
                  pl.BlockSpec((1, ed), lambda i: (0, 0)),
                  pl.BlockSpec((ed, D), lambda i: (0, 0)),
                  pl.BlockSpec((1, D), lambda i: (0, 0))],
        out_specs=pl.BlockSpec((BE, D), lambda i: (i, 0)),
        out_shape=jax.ShapeDtypeStruct((E, D), jnp.float32),
    )(edge_features, W_e, be2, wb, bm1r)

    row = edge_index[0]
    col = edge_index[1]
    zeros = jnp.zeros((N, DP), jnp.float32)

    mesh = plsc.VectorSubcoreMesh(core_axis_name="c", subcore_axis_name="s",
                                  num_cores=NC, num_subcores=NS)
    s2 = pl.kernel(
        _sc_edge_kernel,
        out_type=jax.ShapeDtypeStruct((NC, N, DP), jnp.float32),
        mesh=mesh,
        scratch_types=[
            pltpu.VMEM_SHARED((N, DP), jnp.float32),
            pltpu.VMEM((SUBS, SUB), jnp.int32),
            pltpu.VMEM((SUBS, SUB), jnp.int32),
            pltpu.VMEM((C, D), jnp.float32),
            pltpu.VMEM((C, DP), jnp.float32),
            pltpu.VMEM((C, DP), jnp.float32),
            pltpu.SemaphoreType.DMA,
            pltpu.SemaphoreType.DMA,
        ],
    )(xa, row, col, ebb, zeros)

    out_pad = pl.pallas_call(
        _final_body,
        grid=(N // BN,),
        in_specs=[pl.BlockSpec((NC, BN, DP), lambda i: (0, i, 0)),
                  pl.BlockSpec((BN, D), lambda i: (i, 0)),
                  pl.BlockSpec((D, D), lambda i: (0, 0)),
                  pl.BlockSpec((1, D), lambda i: (0, 0)),
                  pl.BlockSpec((D, D), lambda i: (0, 0)),
                  pl.BlockSpec((D, D), lambda i: (0, 0)),
                  pl.BlockSpec((1, D), lambda i: (0, 0)),
                  pl.BlockSpec((D, 128), lambda i: (0, 0)),
                  pl.BlockSpec((1, 128), lambda i: (0, 0))],
        out_specs=pl.BlockSpec((BN, 128), lambda i: (i, 0)),
        out_shape=jax.ShapeDtypeStruct((N, 128), jnp.float32),
    )(s2, x, Wm2, bm2r, Wu[:D], Wu[D:], bu2, wo_pad, bo_pad)

    return out_pad[:, :og]
